# baseline (speedup 1.0000x reference)
import jax
import jax.numpy as jnp
from jax.experimental import pallas as pl

_NUM_CLASSES = 1000
_SMOOTHING = 0.1
_BATCH = 16384
_CLS_BLOCK = 256


def _body(idx_ref, out_ref):
    sv = jnp.float32(_SMOOTHING / (_NUM_CLASSES - 1))
    hit = jnp.float32(1.0 - _SMOOTHING) + sv
    b = pl.program_id(0)
    classes = (
        jax.lax.broadcasted_iota(jnp.int32, (_CLS_BLOCK, _BATCH), 0)
        + b * _CLS_BLOCK
    )
    out_ref[...] = jnp.where(classes == idx_ref[...], hit, sv)


@jax.jit
def kernel(x_i):
    idx2d = x_i.astype(jnp.int32).reshape(1, _BATCH)
    out_t = pl.pallas_call(
        _body,
        grid=(pl.cdiv(_NUM_CLASSES, _CLS_BLOCK),),
        in_specs=[pl.BlockSpec((1, _BATCH), lambda i: (0, 0))],
        out_specs=pl.BlockSpec((_CLS_BLOCK, _BATCH), lambda i: (i, 0)),
        out_shape=jax.ShapeDtypeStruct((_NUM_CLASSES, _BATCH), jnp.float32),
    )(idx2d)
    return out_t.T


# speedup vs baseline: 1.0230x; 1.0230x over previous
import jax
import jax.numpy as jnp
from jax.experimental import pallas as pl

_NUM_CLASSES = 1000
_SMOOTHING = 0.1
_BATCH = 16384
_CLS_BLOCK = 200


def _body(idx_ref, out_ref):
    sv = jnp.float32(_SMOOTHING / (_NUM_CLASSES - 1))
    hit = jnp.float32(1.0 - _SMOOTHING) + sv
    b = pl.program_id(0)
    classes = (
        jax.lax.broadcasted_iota(jnp.int32, (_CLS_BLOCK, _BATCH), 0)
        + b * _CLS_BLOCK
    )
    out_ref[...] = jnp.where(classes == idx_ref[...], hit, sv)


@jax.jit
def kernel(x_i):
    idx2d = x_i.astype(jnp.int32).reshape(1, _BATCH)
    out_t = pl.pallas_call(
        _body,
        grid=(pl.cdiv(_NUM_CLASSES, _CLS_BLOCK),),
        in_specs=[pl.BlockSpec((1, _BATCH), lambda i: (0, 0))],
        out_specs=pl.BlockSpec((_CLS_BLOCK, _BATCH), lambda i: (i, 0)),
        out_shape=jax.ShapeDtypeStruct((_NUM_CLASSES, _BATCH), jnp.float32),
    )(idx2d)
    return out_t.T
